# baseline (speedup 1.0000x reference)
import functools

import jax
import jax.numpy as jnp
import numpy as np
from jax import lax
from jax.experimental import pallas as pl
from jax.experimental.pallas import tpu as pltpu
from jax.experimental.pallas import tpu_sc as plsc

_MUL = 32
_DIM = 128
_RH = 32
_SQ3 = float(np.sqrt(3.0))
_SCALE = float(1.0 / np.sqrt(2.0 * _MUL) / np.sqrt(_RH))

_NC = 2
_NS = 16


def _perm_planar() -> np.ndarray:
    perm = np.zeros(_DIM, dtype=np.int64)
    perm[:_MUL] = np.arange(_MUL)
    for m in range(3):
        for i in range(_MUL):
            perm[_MUL + _MUL * m + i] = _MUL + 3 * i + m
    return np.eye(_DIM, dtype=np.float32)[perm].T


_P_PLANAR = _perm_planar()
_P_OUT = np.ascontiguousarray(_P_PLANAR.T)


def _equiv_planar(hp, ws, bs, wv):
    inv = float(1.0 / np.sqrt(_MUL))
    s = jnp.dot(hp[:, 0:_MUL], ws, preferred_element_type=jnp.float32) * inv
    s = s + bs
    outs = [s]
    for m in range(3):
        a = _MUL + _MUL * m
        outs.append(jnp.dot(hp[:, a:a + _MUL], wv,
                            preferred_element_type=jnp.float32) * inv)
    return jnp.concatenate(outs, axis=1)


def _pre_body(h_ref, p_ref, ws_ref, bs_ref, wv_ref, esrc_ref, edst_ref,
              hin_ref, fsrc_ref, fdst_ref):
    hp = jnp.dot(h_ref[...], p_ref[...], preferred_element_type=jnp.float32)
    hin_ref[...] = _equiv_planar(hp, ws_ref[...], bs_ref[...], wv_ref[...])
    n = hin_ref.shape[0] // esrc_ref.shape[0]
    boff = lax.broadcasted_iota(jnp.int32, esrc_ref.shape, 0) * n
    fsrc_ref[...] = esrc_ref[...] + boff
    fdst_ref[...] = edst_ref[...] + boff


def _pre_call(h2, p_in, li_ws, li_bs, li_wv, e_src, e_dst):
    bn = h2.shape[0]
    b, e = e_src.shape
    return pl.pallas_call(
        _pre_body,
        out_shape=[
            jax.ShapeDtypeStruct((bn, _DIM), jnp.float32),
            jax.ShapeDtypeStruct((b, e), jnp.int32),
            jax.ShapeDtypeStruct((b, e), jnp.int32),
        ],
    )(h2, p_in, li_ws, li_bs, li_wv, e_src, e_dst)


def _gather_call(fsrc2, hin, chunk, nchunks):
    edges = fsrc2.shape[0] * fsrc2.shape[1] // nchunks
    per_w = edges // (_NC * _NS)
    rows_per_w = fsrc2.shape[1]
    k = per_w // rows_per_w
    base_row = chunk * (edges // rows_per_w)
    mesh = plsc.VectorSubcoreMesh(core_axis_name="c", subcore_axis_name="s",
                                  num_cores=_NC, num_subcores=_NS)

    @functools.partial(
        pl.kernel,
        out_type=jax.ShapeDtypeStruct((edges, _DIM), jnp.float32),
        mesh=mesh,
        scratch_types=[
            pltpu.VMEM((k, rows_per_w), jnp.int32),
            pltpu.VMEM((per_w, _DIM), jnp.float32),
            pltpu.SemaphoreType.DMA,
        ],
    )
    def _gather(idx_hbm, table_hbm, out_hbm, idx_v, rows_v, sem):
        wid = lax.axis_index("s") * _NC + lax.axis_index("c")
        pltpu.sync_copy(idx_hbm.at[pl.ds(base_row + wid * k, k)], idx_v)
        cps = [
            pltpu.async_copy(table_hbm.at[idx_v.at[j]],
                             rows_v.at[pl.ds(j * rows_per_w, rows_per_w)], sem)
            for j in range(k)
        ]
        for c in cps:
            c.wait()
        pltpu.sync_copy(rows_v, out_hbm.at[pl.ds(wid * per_w, per_w)])

    return _gather(fsrc2, hin)


_EBLK = 512


def _msg_body(attr_ref, hsrc_ref, rw1_ref, ws_ref, w4_ref, w3_ref, rrep_ref,
              out_ref):
    att = attr_ref[...]
    r0, r1, r2 = att[:, 0:1], att[:, 1:2], att[:, 2:3]
    rnorm = jnp.sqrt(r0 * r0 + r1 * r1 + r2 * r2) + 1e-8
    inv = 1.0 / rnorm
    y0, y1, y2 = _SQ3 * r0 * inv, _SQ3 * r1 * inv, _SQ3 * r2 * inv
    pre = rnorm * rw1_ref[...]
    hid = pre * jax.nn.sigmoid(pre)
    hs = hsrc_ref[...]
    s = hs[:, 0:_MUL]
    v0 = hs[:, _MUL:2 * _MUL]
    v1 = hs[:, 2 * _MUL:3 * _MUL]
    v2 = hs[:, 3 * _MUL:4 * _MUL]
    inner = (v0 * y0 + v1 * y1 + v2 * y2) * (1.0 / _SQ3)
    hrep = jnp.dot(hid, rrep_ref[...], preferred_element_type=jnp.float32)
    hlo, hhi = hrep[:, :512], hrep[:, 512:]
    bf = jnp.bfloat16

    def contract(t):
        x = hlo * t[:, :512] + hhi * t[:, 512:]
        y = (x[:, :128] + x[:, 128:256]) + (x[:, 256:384] + x[:, 384:512])
        return (y[:, :32] + y[:, 32:64]) + (y[:, 64:96] + y[:, 96:128])

    def bdot(a, w):
        return jnp.dot(a.astype(bf), w,
                       preferred_element_type=jnp.float32)

    ts = bdot(s, ws_ref[...])
    o1 = contract(ts[:, :1024])
    o2 = contract(ts[:, 1024:])
    o4 = contract(bdot(inner, w4_ref[...]))
    o30 = contract(bdot(v0, w3_ref[...]))
    o31 = contract(bdot(v1, w3_ref[...]))
    o32 = contract(bdot(v2, w3_ref[...]))
    m_s = _SCALE * (o1 + o4)
    out_ref[...] = jnp.concatenate(
        [m_s,
         _SCALE * (o2 * y0 + o30),
         _SCALE * (o2 * y1 + o31),
         _SCALE * (o2 * y2 + o32)], axis=1)


def _msg_call(attr2, hsrc, rad_w1, w_s, w_4, w_3, r_rep, chunk, nchunks):
    edges = attr2.shape[0] // nchunks
    grid = edges // _EBLK
    off = chunk * grid
    return pl.pallas_call(
        _msg_body,
        grid=(grid,),
        in_specs=[
            pl.BlockSpec((_EBLK, 3), lambda i: (i + off, 0)),
            pl.BlockSpec((_EBLK, _DIM), lambda i: (i, 0)),
            pl.BlockSpec((1, _RH), lambda i: (0, 0)),
            pl.BlockSpec((_RH, 2048), lambda i: (0, 0)),
            pl.BlockSpec((_RH, 1024), lambda i: (0, 0)),
            pl.BlockSpec((_RH, 1024), lambda i: (0, 0)),
            pl.BlockSpec((_RH, 1024), lambda i: (0, 0)),
        ],
        out_specs=pl.BlockSpec((_EBLK, _DIM), lambda i: (i, 0)),
        out_shape=jax.ShapeDtypeStruct((edges, _DIM), jnp.float32),
    )(attr2, hsrc, rad_w1, w_s, w_4, w_3, r_rep)


def _scatter_call(fdst2, msg, zeros_hbm, chunk, nchunks):
    edges = fdst2.shape[0] * fdst2.shape[1] // nchunks
    bn = zeros_hbm.shape[0]
    per_w = edges // (_NC * _NS)
    rows_per_w = fdst2.shape[1]
    k = per_w // rows_per_w
    base_row = chunk * (edges // rows_per_w)
    rows_per_tile = bn // _NS
    mesh = plsc.VectorSubcoreMesh(core_axis_name="c", subcore_axis_name="s",
                                  num_cores=_NC, num_subcores=_NS)

    @functools.partial(
        pl.kernel,
        out_type=jax.ShapeDtypeStruct((_NC, bn, _DIM), jnp.float32),
        mesh=mesh,
        scratch_types=[
            pltpu.VMEM((k, rows_per_w), jnp.int32),
            pltpu.VMEM((per_w, _DIM), jnp.float32),
            pltpu.VMEM_SHARED((bn, _DIM), jnp.float32),
        ],
    )
    def _scatter(idx_hbm, msg_hbm, z_hbm, out_hbm, idx_v, rows_v, acc):
        cid = lax.axis_index("c")
        sid = lax.axis_index("s")
        wid = sid * _NC + cid
        r0 = sid * rows_per_tile
        pltpu.sync_copy(z_hbm.at[pl.ds(r0, rows_per_tile)],
                        acc.at[pl.ds(r0, rows_per_tile)])
        pltpu.sync_copy(idx_hbm.at[pl.ds(base_row + wid * k, k)], idx_v)
        pltpu.sync_copy(msg_hbm.at[pl.ds(wid * per_w, per_w)], rows_v)
        plsc.subcore_barrier()
        for j in range(k):
            pltpu.sync_copy(rows_v.at[pl.ds(j * rows_per_w, rows_per_w)],
                            acc.at[idx_v.at[j]], add=True)
        plsc.subcore_barrier()
        pltpu.sync_copy(acc.at[pl.ds(r0, rows_per_tile)],
                        out_hbm.at[cid, pl.ds(r0, rows_per_tile)])

    return _scatter(fdst2, msg, zeros_hbm)


def _fin_body(*refs):
    part_refs = refs[:-7]
    h_ref, p_ref, ws_ref, bs_ref, wv_ref, pout_ref, out_ref = refs[-7:]
    hp = jnp.dot(h_ref[...], p_ref[...], preferred_element_type=jnp.float32)
    res = _equiv_planar(hp, ws_ref[...], bs_ref[...], wv_ref[...])
    x = res
    for pr in part_refs:
        x = x + pr[0] + pr[1]
    s = x[:, 0:_MUL]
    v0 = x[:, _MUL:2 * _MUL]
    v1 = x[:, 2 * _MUL:3 * _MUL]
    v2 = x[:, 3 * _MUL:4 * _MUL]
    ns = jnp.abs(s)
    s2 = s * (ns * jax.nn.sigmoid(ns)) / (ns + 1e-8)
    nv = jnp.sqrt(v0 * v0 + v1 * v1 + v2 * v2)
    f = (nv * jax.nn.sigmoid(nv)) / (nv + 1e-8)
    y = jnp.concatenate([s2, v0 * f, v1 * f, v2 * f], axis=1)
    out_ref[...] = jnp.dot(y, pout_ref[...],
                           preferred_element_type=jnp.float32)


def _fin_call(parts_list, h2, p_in, res_ws, res_bs, res_wv, pout):
    bn = h2.shape[0]
    return pl.pallas_call(
        _fin_body,
        out_shape=jax.ShapeDtypeStruct((bn, _DIM), jnp.float32),
    )(*parts_list, h2, p_in, res_ws, res_bs, res_wv, pout)


def kernel(h, e_src, e_dst, e_attr3, li_ws, li_bs, li_wv,
           rad_w1, rad_w2, res_ws, res_bs, res_wv):
    b, n, c = h.shape
    e = e_attr3.shape[1]
    edges = b * e

    p_in = jnp.asarray(_P_PLANAR)
    rad4 = rad_w2.reshape(_RH, 4, _MUL, _MUL)
    def perm(p):
        return rad4[:, p].transpose(1, 0, 2).reshape(_MUL, _RH * _MUL)
    w_s = jnp.concatenate([perm(0), perm(1)], axis=1).astype(jnp.bfloat16)
    w_4 = perm(3).astype(jnp.bfloat16)
    w_3 = perm(2).astype(jnp.bfloat16)
    r_rep = jnp.asarray(np.repeat(np.eye(_RH, dtype=np.float32), _MUL, axis=1))
    pout = jnp.asarray(_P_OUT)
    zeros = jnp.zeros((b * n, _DIM), jnp.float32)

    h2 = h.reshape(b * n, c)
    hin, fsrc, fdst = _pre_call(h2, p_in, li_ws, li_bs.reshape(1, _MUL),
                                li_wv, e_src, e_dst)
    idx_cols = 128
    fsrc2 = fsrc.reshape(edges // idx_cols, idx_cols)
    fdst2 = fdst.reshape(edges // idx_cols, idx_cols)
    attr2 = e_attr3.reshape(edges, 3)
    nchunks = 4
    parts_list = []
    for ck in range(nchunks):
        hsrc = _gather_call(fsrc2, hin, ck, nchunks)
        msg = _msg_call(attr2, hsrc, rad_w1, w_s, w_4, w_3, r_rep,
                        ck, nchunks)
        parts_list.append(_scatter_call(fdst2, msg, zeros, ck, nchunks))
    out = _fin_call(parts_list, h2, p_in, res_ws, res_bs.reshape(1, _MUL),
                    res_wv, pout)
    return out.reshape(b, n, c)


# speedup vs baseline: 1.1467x; 1.1467x over previous
import functools

import jax
import jax.numpy as jnp
import numpy as np
from jax import lax
from jax.experimental import pallas as pl
from jax.experimental.pallas import tpu as pltpu
from jax.experimental.pallas import tpu_sc as plsc

_MUL = 32
_DIM = 128
_RH = 32
_SQ3 = float(np.sqrt(3.0))
_SCALE = float(1.0 / np.sqrt(2.0 * _MUL) / np.sqrt(_RH))

_NC = 2
_NS = 16


def _perm_planar() -> np.ndarray:
    perm = np.zeros(_DIM, dtype=np.int64)
    perm[:_MUL] = np.arange(_MUL)
    for m in range(3):
        for i in range(_MUL):
            perm[_MUL + _MUL * m + i] = _MUL + 3 * i + m
    return np.eye(_DIM, dtype=np.float32)[perm].T


_P_PLANAR = _perm_planar()
_P_OUT = np.ascontiguousarray(_P_PLANAR.T)


def _equiv_planar(hp, ws, bs, wv):
    inv = float(1.0 / np.sqrt(_MUL))
    s = jnp.dot(hp[:, 0:_MUL], ws, preferred_element_type=jnp.float32) * inv
    s = s + bs
    outs = [s]
    for m in range(3):
        a = _MUL + _MUL * m
        outs.append(jnp.dot(hp[:, a:a + _MUL], wv,
                            preferred_element_type=jnp.float32) * inv)
    return jnp.concatenate(outs, axis=1)


def _pre_body(h_ref, p_ref, ws_ref, bs_ref, wv_ref, esrc_ref, edst_ref,
              hin_ref, fsrc_ref, fdst_ref):
    hp = jnp.dot(h_ref[...], p_ref[...], preferred_element_type=jnp.float32)
    hin_ref[...] = _equiv_planar(hp, ws_ref[...], bs_ref[...], wv_ref[...])
    n = hin_ref.shape[0] // esrc_ref.shape[0]
    boff = lax.broadcasted_iota(jnp.int32, esrc_ref.shape, 0) * n
    fsrc_ref[...] = esrc_ref[...] + boff
    fdst_ref[...] = edst_ref[...] + boff


def _pre_call(h2, p_in, li_ws, li_bs, li_wv, e_src, e_dst):
    bn = h2.shape[0]
    b, e = e_src.shape
    return pl.pallas_call(
        _pre_body,
        out_shape=[
            jax.ShapeDtypeStruct((bn, _DIM), jnp.float32),
            jax.ShapeDtypeStruct((b, e), jnp.int32),
            jax.ShapeDtypeStruct((b, e), jnp.int32),
        ],
    )(h2, p_in, li_ws, li_bs, li_wv, e_src, e_dst)


def _gather_call(fsrc2, hin, chunk, nchunks):
    edges = fsrc2.shape[0] * fsrc2.shape[1] // nchunks
    per_w = edges // (_NC * _NS)
    rows_per_w = fsrc2.shape[1]
    k = per_w // rows_per_w
    base_row = chunk * (edges // rows_per_w)
    mesh = plsc.VectorSubcoreMesh(core_axis_name="c", subcore_axis_name="s",
                                  num_cores=_NC, num_subcores=_NS)

    @functools.partial(
        pl.kernel,
        out_type=jax.ShapeDtypeStruct((edges, _DIM), jnp.float32),
        mesh=mesh,
        scratch_types=[
            pltpu.VMEM((k, rows_per_w), jnp.int32),
            pltpu.VMEM((per_w, _DIM), jnp.float32),
            pltpu.SemaphoreType.DMA,
        ],
    )
    def _gather(idx_hbm, table_hbm, out_hbm, idx_v, rows_v, sem):
        wid = lax.axis_index("s") * _NC + lax.axis_index("c")
        pltpu.sync_copy(idx_hbm.at[pl.ds(base_row + wid * k, k)], idx_v)
        cps = [
            pltpu.async_copy(table_hbm.at[idx_v.at[j]],
                             rows_v.at[pl.ds(j * rows_per_w, rows_per_w)], sem)
            for j in range(k)
        ]
        for c in cps:
            c.wait()
        pltpu.sync_copy(rows_v, out_hbm.at[pl.ds(wid * per_w, per_w)])

    return _gather(fsrc2, hin)


_EBLK = 1024


def _msg_body(attr_ref, hsrc_ref, rw1_ref, wbig_ref, rrep_ref, out_ref):
    att = attr_ref[...]
    r0, r1, r2 = att[:, 0:1], att[:, 1:2], att[:, 2:3]
    rnorm = jnp.sqrt(r0 * r0 + r1 * r1 + r2 * r2) + 1e-8
    inv = 1.0 / rnorm
    y0, y1, y2 = _SQ3 * r0 * inv, _SQ3 * r1 * inv, _SQ3 * r2 * inv
    pre = rnorm * rw1_ref[...]
    hid = pre * jax.nn.sigmoid(pre)
    hs = hsrc_ref[...]
    s = hs[:, 0:_MUL]
    v0 = hs[:, _MUL:2 * _MUL]
    v1 = hs[:, 2 * _MUL:3 * _MUL]
    v2 = hs[:, 3 * _MUL:4 * _MUL]
    inner = (v0 * y0 + v1 * y1 + v2 * y2) * (1.0 / _SQ3)
    hrep = jnp.dot(hid, rrep_ref[...], preferred_element_type=jnp.float32)
    hlo, hhi = hrep[:, :512], hrep[:, 512:]
    bf = jnp.bfloat16

    def contract(t):
        x = hlo * t[:, :512] + hhi * t[:, 512:]
        y = (x[:, :128] + x[:, 128:256]) + (x[:, 256:384] + x[:, 384:512])
        return (y[:, :32] + y[:, 32:64]) + (y[:, 64:96] + y[:, 96:128])

    x_all = jnp.concatenate(
        [s, inner, v0, s * y0, v1, s * y1, v2, s * y2], axis=1)
    t_all = jnp.dot(x_all.astype(bf), wbig_ref[...],
                    preferred_element_type=jnp.float32)
    out_ref[...] = jnp.concatenate(
        [_SCALE * contract(t_all[:, :1024]),
         _SCALE * contract(t_all[:, 1024:2048]),
         _SCALE * contract(t_all[:, 2048:3072]),
         _SCALE * contract(t_all[:, 3072:])], axis=1)


def _msg_call(attr2, hsrc, rad_w1, w_big, r_rep, chunk, nchunks):
    edges = attr2.shape[0] // nchunks
    grid = edges // _EBLK
    off = chunk * grid
    return pl.pallas_call(
        _msg_body,
        grid=(grid,),
        in_specs=[
            pl.BlockSpec((_EBLK, 3), lambda i: (i + off, 0)),
            pl.BlockSpec((_EBLK, _DIM), lambda i: (i, 0)),
            pl.BlockSpec((1, _RH), lambda i: (0, 0)),
            pl.BlockSpec((256, 4096), lambda i: (0, 0)),
            pl.BlockSpec((_RH, 1024), lambda i: (0, 0)),
        ],
        out_specs=pl.BlockSpec((_EBLK, _DIM), lambda i: (i, 0)),
        out_shape=jax.ShapeDtypeStruct((edges, _DIM), jnp.float32),
    )(attr2, hsrc, rad_w1, w_big, r_rep)


def _scatter_call(fdst2, msg, zeros_hbm, chunk, nchunks):
    edges = fdst2.shape[0] * fdst2.shape[1] // nchunks
    bn = zeros_hbm.shape[0]
    per_w = edges // (_NC * _NS)
    rows_per_w = fdst2.shape[1]
    k = per_w // rows_per_w
    base_row = chunk * (edges // rows_per_w)
    rows_per_tile = bn // _NS
    mesh = plsc.VectorSubcoreMesh(core_axis_name="c", subcore_axis_name="s",
                                  num_cores=_NC, num_subcores=_NS)

    @functools.partial(
        pl.kernel,
        out_type=jax.ShapeDtypeStruct((_NC, bn, _DIM), jnp.float32),
        mesh=mesh,
        scratch_types=[
            pltpu.VMEM((k, rows_per_w), jnp.int32),
            pltpu.VMEM((per_w, _DIM), jnp.float32),
            pltpu.VMEM_SHARED((bn, _DIM), jnp.float32),
        ],
    )
    def _scatter(idx_hbm, msg_hbm, z_hbm, out_hbm, idx_v, rows_v, acc):
        cid = lax.axis_index("c")
        sid = lax.axis_index("s")
        wid = sid * _NC + cid
        r0 = sid * rows_per_tile
        pltpu.sync_copy(z_hbm.at[pl.ds(r0, rows_per_tile)],
                        acc.at[pl.ds(r0, rows_per_tile)])
        pltpu.sync_copy(idx_hbm.at[pl.ds(base_row + wid * k, k)], idx_v)
        pltpu.sync_copy(msg_hbm.at[pl.ds(wid * per_w, per_w)], rows_v)
        plsc.subcore_barrier()
        for j in range(k):
            pltpu.sync_copy(rows_v.at[pl.ds(j * rows_per_w, rows_per_w)],
                            acc.at[idx_v.at[j]], add=True)
        plsc.subcore_barrier()
        pltpu.sync_copy(acc.at[pl.ds(r0, rows_per_tile)],
                        out_hbm.at[cid, pl.ds(r0, rows_per_tile)])

    return _scatter(fdst2, msg, zeros_hbm)


def _fin_body(*refs):
    part_refs = refs[:-7]
    h_ref, p_ref, ws_ref, bs_ref, wv_ref, pout_ref, out_ref = refs[-7:]
    hp = jnp.dot(h_ref[...], p_ref[...], preferred_element_type=jnp.float32)
    res = _equiv_planar(hp, ws_ref[...], bs_ref[...], wv_ref[...])
    x = res
    for pr in part_refs:
        x = x + pr[0] + pr[1]
    s = x[:, 0:_MUL]
    v0 = x[:, _MUL:2 * _MUL]
    v1 = x[:, 2 * _MUL:3 * _MUL]
    v2 = x[:, 3 * _MUL:4 * _MUL]
    ns = jnp.abs(s)
    s2 = s * (ns * jax.nn.sigmoid(ns)) / (ns + 1e-8)
    nv = jnp.sqrt(v0 * v0 + v1 * v1 + v2 * v2)
    f = (nv * jax.nn.sigmoid(nv)) / (nv + 1e-8)
    y = jnp.concatenate([s2, v0 * f, v1 * f, v2 * f], axis=1)
    out_ref[...] = jnp.dot(y, pout_ref[...],
                           preferred_element_type=jnp.float32)


def _fin_call(parts_list, h2, p_in, res_ws, res_bs, res_wv, pout):
    bn = h2.shape[0]
    return pl.pallas_call(
        _fin_body,
        out_shape=jax.ShapeDtypeStruct((bn, _DIM), jnp.float32),
    )(*parts_list, h2, p_in, res_ws, res_bs, res_wv, pout)


def kernel(h, e_src, e_dst, e_attr3, li_ws, li_bs, li_wv,
           rad_w1, rad_w2, res_ws, res_bs, res_wv):
    b, n, c = h.shape
    e = e_attr3.shape[1]
    edges = b * e

    p_in = jnp.asarray(_P_PLANAR)
    rad4 = rad_w2.reshape(_RH, 4, _MUL, _MUL)
    def perm(p):
        return rad4[:, p].transpose(1, 0, 2).reshape(_MUL, _RH * _MUL)
    z = jnp.zeros((_MUL, _RH * _MUL), jnp.float32)
    w_mv = [perm(2), perm(1)]
    rows = []
    for blk_i, pair in enumerate([[perm(0), perm(3)], w_mv, w_mv, w_mv]):
        for half in range(2):
            rows.append(jnp.concatenate(
                [pair[half] if cb == blk_i else z for cb in range(4)], axis=1))
    w_big = jnp.concatenate(rows, axis=0).astype(jnp.bfloat16)
    r_rep = jnp.asarray(np.repeat(np.eye(_RH, dtype=np.float32), _MUL, axis=1))
    pout = jnp.asarray(_P_OUT)
    zeros = jnp.zeros((b * n, _DIM), jnp.float32)

    h2 = h.reshape(b * n, c)
    hin, fsrc, fdst = _pre_call(h2, p_in, li_ws, li_bs.reshape(1, _MUL),
                                li_wv, e_src, e_dst)
    idx_cols = 128
    fsrc2 = fsrc.reshape(edges // idx_cols, idx_cols)
    fdst2 = fdst.reshape(edges // idx_cols, idx_cols)
    attr2 = e_attr3.reshape(edges, 3)
    nchunks = 2
    parts_list = []
    for ck in range(nchunks):
        hsrc = _gather_call(fsrc2, hin, ck, nchunks)
        msg = _msg_call(attr2, hsrc, rad_w1, w_big, r_rep,
                        ck, nchunks)
        parts_list.append(_scatter_call(fdst2, msg, zeros, ck, nchunks))
    out = _fin_call(parts_list, h2, p_in, res_ws, res_bs.reshape(1, _MUL),
                    res_wv, pout)
    return out.reshape(b, n, c)


# speedup vs baseline: 1.1543x; 1.0066x over previous
import functools

import jax
import jax.numpy as jnp
import numpy as np
from jax import lax
from jax.experimental import pallas as pl
from jax.experimental.pallas import tpu as pltpu
from jax.experimental.pallas import tpu_sc as plsc

_MUL = 32
_DIM = 128
_RH = 32
_SQ3 = float(np.sqrt(3.0))
_SCALE = float(1.0 / np.sqrt(2.0 * _MUL) / np.sqrt(_RH))

_NC = 2
_NS = 16


def _perm_planar() -> np.ndarray:
    perm = np.zeros(_DIM, dtype=np.int64)
    perm[:_MUL] = np.arange(_MUL)
    for m in range(3):
        for i in range(_MUL):
            perm[_MUL + _MUL * m + i] = _MUL + 3 * i + m
    return np.eye(_DIM, dtype=np.float32)[perm].T


_P_PLANAR = _perm_planar()
_P_OUT = np.ascontiguousarray(_P_PLANAR.T)


def _equiv_planar(hp, ws, bs, wv):
    inv = float(1.0 / np.sqrt(_MUL))
    s = jnp.dot(hp[:, 0:_MUL], ws, preferred_element_type=jnp.float32) * inv
    s = s + bs
    outs = [s]
    for m in range(3):
        a = _MUL + _MUL * m
        outs.append(jnp.dot(hp[:, a:a + _MUL], wv,
                            preferred_element_type=jnp.float32) * inv)
    return jnp.concatenate(outs, axis=1)


def _pre_body(h_ref, p_ref, ws_ref, bs_ref, wv_ref, esrc_ref, edst_ref,
              hin_ref, fsrc_ref, fdst_ref):
    hp = jnp.dot(h_ref[...], p_ref[...], preferred_element_type=jnp.float32)
    hin_ref[...] = _equiv_planar(hp, ws_ref[...], bs_ref[...], wv_ref[...])
    n = hin_ref.shape[0] // esrc_ref.shape[0]
    boff = lax.broadcasted_iota(jnp.int32, esrc_ref.shape, 0) * n
    fsrc_ref[...] = esrc_ref[...] + boff
    fdst_ref[...] = edst_ref[...] + boff


def _pre_call(h2, p_in, li_ws, li_bs, li_wv, e_src, e_dst):
    bn = h2.shape[0]
    b, e = e_src.shape
    return pl.pallas_call(
        _pre_body,
        out_shape=[
            jax.ShapeDtypeStruct((bn, _DIM), jnp.float32),
            jax.ShapeDtypeStruct((b, e), jnp.int32),
            jax.ShapeDtypeStruct((b, e), jnp.int32),
        ],
    )(h2, p_in, li_ws, li_bs, li_wv, e_src, e_dst)


def _gather_call(fsrc2, hin, chunk, nchunks):
    edges = fsrc2.shape[0] * fsrc2.shape[1] // nchunks
    per_w = edges // (_NC * _NS)
    rows_per_w = fsrc2.shape[1]
    k = per_w // rows_per_w
    base_row = chunk * (edges // rows_per_w)
    mesh = plsc.VectorSubcoreMesh(core_axis_name="c", subcore_axis_name="s",
                                  num_cores=_NC, num_subcores=_NS)

    @functools.partial(
        pl.kernel,
        out_type=jax.ShapeDtypeStruct((edges, _DIM), jnp.float32),
        mesh=mesh,
        scratch_types=[
            pltpu.VMEM((k, rows_per_w), jnp.int32),
            pltpu.VMEM((per_w, _DIM), jnp.float32),
            pltpu.SemaphoreType.DMA,
        ],
    )
    def _gather(idx_hbm, table_hbm, out_hbm, idx_v, rows_v, sem):
        wid = lax.axis_index("s") * _NC + lax.axis_index("c")
        pltpu.sync_copy(idx_hbm.at[pl.ds(base_row + wid * k, k)], idx_v)
        cps = [
            pltpu.async_copy(table_hbm.at[idx_v.at[j]],
                             rows_v.at[pl.ds(j * rows_per_w, rows_per_w)], sem)
            for j in range(k)
        ]
        for c in cps:
            c.wait()
        pltpu.sync_copy(rows_v, out_hbm.at[pl.ds(wid * per_w, per_w)])

    return _gather(fsrc2, hin)


_EBLK = 1024


def _msg_body(attr_ref, hsrc_ref, rw1_ref, wbig_ref, rrep_ref, out_ref):
    att = attr_ref[...]
    r0, r1, r2 = att[:, 0:1], att[:, 1:2], att[:, 2:3]
    rnorm = jnp.sqrt(r0 * r0 + r1 * r1 + r2 * r2) + 1e-8
    inv = 1.0 / rnorm
    y0, y1, y2 = _SQ3 * r0 * inv, _SQ3 * r1 * inv, _SQ3 * r2 * inv
    pre = rnorm * rw1_ref[...]
    hid = pre * jax.nn.sigmoid(pre)
    hs = hsrc_ref[...]
    s = hs[:, 0:_MUL]
    v0 = hs[:, _MUL:2 * _MUL]
    v1 = hs[:, 2 * _MUL:3 * _MUL]
    v2 = hs[:, 3 * _MUL:4 * _MUL]
    inner = (v0 * y0 + v1 * y1 + v2 * y2) * (1.0 / _SQ3)
    hrep = jnp.dot(hid, rrep_ref[...], preferred_element_type=jnp.float32)
    hlo, hhi = hrep[:, :512], hrep[:, 512:]
    bf = jnp.bfloat16

    def contract(t):
        x = hlo * t[:, :512] + hhi * t[:, 512:]
        y = (x[:, :128] + x[:, 128:256]) + (x[:, 256:384] + x[:, 384:512])
        return (y[:, :32] + y[:, 32:64]) + (y[:, 64:96] + y[:, 96:128])

    x_all = jnp.concatenate(
        [s, inner, v0, s * y0, v1, s * y1, v2, s * y2], axis=1)
    t_all = jnp.dot(x_all.astype(bf), wbig_ref[...],
                    preferred_element_type=jnp.float32)
    out_ref[...] = jnp.concatenate(
        [_SCALE * contract(t_all[:, :1024]),
         _SCALE * contract(t_all[:, 1024:2048]),
         _SCALE * contract(t_all[:, 2048:3072]),
         _SCALE * contract(t_all[:, 3072:])], axis=1)


def _msg_call(attr2, hsrc, rad_w1, w_big, r_rep, chunk, nchunks):
    edges = attr2.shape[0] // nchunks
    grid = edges // _EBLK
    off = chunk * grid
    return pl.pallas_call(
        _msg_body,
        grid=(grid,),
        in_specs=[
            pl.BlockSpec((_EBLK, 3), lambda i: (i + off, 0)),
            pl.BlockSpec((_EBLK, _DIM), lambda i: (i, 0)),
            pl.BlockSpec((1, _RH), lambda i: (0, 0)),
            pl.BlockSpec((256, 4096), lambda i: (0, 0)),
            pl.BlockSpec((_RH, 1024), lambda i: (0, 0)),
        ],
        out_specs=pl.BlockSpec((_EBLK, _DIM), lambda i: (i, 0)),
        out_shape=jax.ShapeDtypeStruct((edges, _DIM), jnp.float32),
    )(attr2, hsrc, rad_w1, w_big, r_rep)


def _scatter_call(fdst2, msg, zeros_hbm, chunk, nchunks):
    edges = fdst2.shape[0] * fdst2.shape[1] // nchunks
    bn = zeros_hbm.shape[0]
    per_w = edges // (_NC * _NS)
    rows_per_w = fdst2.shape[1]
    k = per_w // rows_per_w
    base_row = chunk * (edges // rows_per_w)
    rows_per_tile = bn // _NS
    mesh = plsc.VectorSubcoreMesh(core_axis_name="c", subcore_axis_name="s",
                                  num_cores=_NC, num_subcores=_NS)

    @functools.partial(
        pl.kernel,
        out_type=jax.ShapeDtypeStruct((_NC, bn, _DIM), jnp.float32),
        mesh=mesh,
        scratch_types=[
            pltpu.VMEM((k, rows_per_w), jnp.int32),
            pltpu.VMEM((per_w, _DIM), jnp.float32),
            pltpu.VMEM_SHARED((bn, _DIM), jnp.float32),
        ],
    )
    def _scatter(idx_hbm, msg_hbm, z_hbm, out_hbm, idx_v, rows_v, acc):
        cid = lax.axis_index("c")
        sid = lax.axis_index("s")
        wid = sid * _NC + cid
        r0 = sid * rows_per_tile
        pltpu.sync_copy(z_hbm.at[pl.ds(r0, rows_per_tile)],
                        acc.at[pl.ds(r0, rows_per_tile)])
        pltpu.sync_copy(idx_hbm.at[pl.ds(base_row + wid * k, k)], idx_v)
        pltpu.sync_copy(msg_hbm.at[pl.ds(wid * per_w, per_w)], rows_v)
        plsc.subcore_barrier()
        for j in range(k):
            pltpu.sync_copy(rows_v.at[pl.ds(j * rows_per_w, rows_per_w)],
                            acc.at[idx_v.at[j]], add=True)
        plsc.subcore_barrier()
        pltpu.sync_copy(acc.at[pl.ds(r0, rows_per_tile)],
                        out_hbm.at[cid, pl.ds(r0, rows_per_tile)])

    return _scatter(fdst2, msg, zeros_hbm)


def _fin_body(*refs):
    part_refs = refs[:-7]
    h_ref, p_ref, ws_ref, bs_ref, wv_ref, pout_ref, out_ref = refs[-7:]
    hp = jnp.dot(h_ref[...], p_ref[...], preferred_element_type=jnp.float32)
    res = _equiv_planar(hp, ws_ref[...], bs_ref[...], wv_ref[...])
    x = res
    for pr in part_refs:
        x = x + pr[0] + pr[1]
    s = x[:, 0:_MUL]
    v0 = x[:, _MUL:2 * _MUL]
    v1 = x[:, 2 * _MUL:3 * _MUL]
    v2 = x[:, 3 * _MUL:4 * _MUL]
    ns = jnp.abs(s)
    s2 = s * (ns * jax.nn.sigmoid(ns)) / (ns + 1e-8)
    nv = jnp.sqrt(v0 * v0 + v1 * v1 + v2 * v2)
    f = (nv * jax.nn.sigmoid(nv)) / (nv + 1e-8)
    y = jnp.concatenate([s2, v0 * f, v1 * f, v2 * f], axis=1)
    out_ref[...] = jnp.dot(y, pout_ref[...],
                           preferred_element_type=jnp.float32)


def _fin_call(parts_list, h2, p_in, res_ws, res_bs, res_wv, pout):
    bn = h2.shape[0]
    return pl.pallas_call(
        _fin_body,
        out_shape=jax.ShapeDtypeStruct((bn, _DIM), jnp.float32),
    )(*parts_list, h2, p_in, res_ws, res_bs, res_wv, pout)


def kernel(h, e_src, e_dst, e_attr3, li_ws, li_bs, li_wv,
           rad_w1, rad_w2, res_ws, res_bs, res_wv):
    b, n, c = h.shape
    e = e_attr3.shape[1]
    edges = b * e

    p_in = jnp.asarray(_P_PLANAR)
    rad4 = rad_w2.reshape(_RH, 4, _MUL, _MUL)
    def perm(p):
        return rad4[:, p].transpose(1, 0, 2).reshape(_MUL, _RH * _MUL)
    z = jnp.zeros((_MUL, _RH * _MUL), jnp.float32)
    w_mv = [perm(2), perm(1)]
    rows = []
    for blk_i, pair in enumerate([[perm(0), perm(3)], w_mv, w_mv, w_mv]):
        for half in range(2):
            rows.append(jnp.concatenate(
                [pair[half] if cb == blk_i else z for cb in range(4)], axis=1))
    w_big = jnp.concatenate(rows, axis=0).astype(jnp.bfloat16)
    r_rep = jnp.asarray(np.repeat(np.eye(_RH, dtype=np.float32), _MUL, axis=1))
    pout = jnp.asarray(_P_OUT)
    zeros = jnp.zeros((b * n, _DIM), jnp.float32)

    h2 = h.reshape(b * n, c)
    hin, fsrc, fdst = _pre_call(h2, p_in, li_ws, li_bs.reshape(1, _MUL),
                                li_wv, e_src, e_dst)
    idx_cols = 128
    fsrc2 = fsrc.reshape(edges // idx_cols, idx_cols)
    fdst2 = fdst.reshape(edges // idx_cols, idx_cols)
    attr2 = e_attr3.reshape(edges, 3)
    nchunks = 1
    parts_list = []
    for ck in range(nchunks):
        hsrc = _gather_call(fsrc2, hin, ck, nchunks)
        msg = _msg_call(attr2, hsrc, rad_w1, w_big, r_rep,
                        ck, nchunks)
        parts_list.append(_scatter_call(fdst2, msg, zeros, ck, nchunks))
    out = _fin_call(parts_list, h2, p_in, res_ws, res_bs.reshape(1, _MUL),
                    res_wv, pout)
    return out.reshape(b, n, c)


# speedup vs baseline: 2.8529x; 2.4716x over previous
import functools

import jax
import jax.numpy as jnp
import numpy as np
from jax import lax
from jax.experimental import pallas as pl
from jax.experimental.pallas import tpu as pltpu
from jax.experimental.pallas import tpu_sc as plsc

_MUL = 32
_DIM = 128
_RH = 32
_SQ3 = float(np.sqrt(3.0))
_SCALE = float(1.0 / np.sqrt(2.0 * _MUL) / np.sqrt(_RH))

_NC = 2
_NS = 16


def _perm_planar() -> np.ndarray:
    perm = np.zeros(_DIM, dtype=np.int64)
    perm[:_MUL] = np.arange(_MUL)
    for m in range(3):
        for i in range(_MUL):
            perm[_MUL + _MUL * m + i] = _MUL + 3 * i + m
    return np.eye(_DIM, dtype=np.float32)[perm].T


_P_PLANAR = _perm_planar()
_P_OUT = np.ascontiguousarray(_P_PLANAR.T)


def _equiv_planar(hp, ws, bs, wv):
    inv = float(1.0 / np.sqrt(_MUL))
    s = jnp.dot(hp[:, 0:_MUL], ws, preferred_element_type=jnp.float32) * inv
    s = s + bs
    outs = [s]
    for m in range(3):
        a = _MUL + _MUL * m
        outs.append(jnp.dot(hp[:, a:a + _MUL], wv,
                            preferred_element_type=jnp.float32) * inv)
    return jnp.concatenate(outs, axis=1)


def _pre_body(h_ref, p_ref, ws_ref, bs_ref, wv_ref, esrc_ref, edst_ref,
              hin_ref, fsrc_ref, fdst_ref):
    hp = jnp.dot(h_ref[...], p_ref[...], preferred_element_type=jnp.float32)
    hin_ref[...] = _equiv_planar(hp, ws_ref[...], bs_ref[...], wv_ref[...])
    n = hin_ref.shape[0] // esrc_ref.shape[0]
    boff = lax.broadcasted_iota(jnp.int32, esrc_ref.shape, 0) * n
    fsrc_ref[...] = esrc_ref[...] + boff
    fdst_ref[...] = edst_ref[...] + boff


def _pre_call(h2, p_in, li_ws, li_bs, li_wv, e_src, e_dst):
    bn = h2.shape[0]
    b, e = e_src.shape
    return pl.pallas_call(
        _pre_body,
        out_shape=[
            jax.ShapeDtypeStruct((bn, _DIM), jnp.float32),
            jax.ShapeDtypeStruct((b, e), jnp.int32),
            jax.ShapeDtypeStruct((b, e), jnp.int32),
        ],
    )(h2, p_in, li_ws, li_bs, li_wv, e_src, e_dst)


def _gather_call(fsrc2, hin, chunk, nchunks):
    edges = fsrc2.shape[0] * fsrc2.shape[1] // nchunks
    per_w = edges // (_NC * _NS)
    rows_per_w = fsrc2.shape[1]
    k = per_w // rows_per_w
    base_row = chunk * (edges // rows_per_w)
    mesh = plsc.VectorSubcoreMesh(core_axis_name="c", subcore_axis_name="s",
                                  num_cores=_NC, num_subcores=_NS)

    @functools.partial(
        pl.kernel,
        out_type=jax.ShapeDtypeStruct((edges, _DIM), jnp.float32),
        mesh=mesh,
        scratch_types=[
            pltpu.VMEM((k, rows_per_w), jnp.int32),
            pltpu.VMEM((per_w, _DIM), jnp.float32),
            pltpu.SemaphoreType.DMA,
        ],
    )
    def _gather(idx_hbm, table_hbm, out_hbm, idx_v, rows_v, sem):
        wid = lax.axis_index("s") * _NC + lax.axis_index("c")
        pltpu.sync_copy(idx_hbm.at[pl.ds(base_row + wid * k, k)], idx_v)
        cps = [
            pltpu.async_copy(table_hbm.at[idx_v.at[j]],
                             rows_v.at[pl.ds(j * rows_per_w, rows_per_w)], sem)
            for j in range(k)
        ]
        for c in cps:
            c.wait()
        pltpu.sync_copy(rows_v, out_hbm.at[pl.ds(wid * per_w, per_w)])

    return _gather(fsrc2, hin)


_EBLK = 1024


def _msg_body(attr_ref, hsrc_ref, rw1_ref, wbig_ref, rrep_ref, out_ref):
    att = attr_ref[...]
    a0, a1, a2 = att[0:1, :], att[1:2, :], att[2:3, :]
    rn = jnp.sqrt(a0 * a0 + a1 * a1 + a2 * a2) + 1e-8
    ir = 1.0 / rn
    zt = jnp.transpose(jnp.concatenate(
        [_SQ3 * a0 * ir, _SQ3 * a1 * ir, _SQ3 * a2 * ir, rn], axis=0))
    y0, y1, y2 = zt[:, 0:1], zt[:, 1:2], zt[:, 2:3]
    rnorm = zt[:, 3:4]
    pre = rnorm * rw1_ref[...]
    hid = pre * jax.nn.sigmoid(pre)
    hs = hsrc_ref[...]
    s = hs[:, 0:_MUL]
    v0 = hs[:, _MUL:2 * _MUL]
    v1 = hs[:, 2 * _MUL:3 * _MUL]
    v2 = hs[:, 3 * _MUL:4 * _MUL]
    inner = (v0 * y0 + v1 * y1 + v2 * y2) * (1.0 / _SQ3)
    hrep = jnp.dot(hid, rrep_ref[...], preferred_element_type=jnp.float32)
    hlo, hhi = hrep[:, :512], hrep[:, 512:]
    bf = jnp.bfloat16

    def contract(t):
        x = hlo * t[:, :512] + hhi * t[:, 512:]
        y = (x[:, :128] + x[:, 128:256]) + (x[:, 256:384] + x[:, 384:512])
        return (y[:, :32] + y[:, 32:64]) + (y[:, 64:96] + y[:, 96:128])

    x_all = jnp.concatenate(
        [s, inner, v0, s * y0, v1, s * y1, v2, s * y2], axis=1)
    t_all = jnp.dot(x_all.astype(bf), wbig_ref[...],
                    preferred_element_type=jnp.float32)
    out_ref[...] = jnp.concatenate(
        [_SCALE * contract(t_all[:, :1024]),
         _SCALE * contract(t_all[:, 1024:2048]),
         _SCALE * contract(t_all[:, 2048:3072]),
         _SCALE * contract(t_all[:, 3072:])], axis=1)


def _msg_call(attr2, hsrc, rad_w1, w_big, r_rep, chunk, nchunks):
    edges = attr2.shape[0] // nchunks
    grid = edges // _EBLK
    off = chunk * grid
    return pl.pallas_call(
        _msg_body,
        grid=(grid,),
        in_specs=[
            pl.BlockSpec((3, _EBLK), lambda i: (0, i + off)),
            pl.BlockSpec((_EBLK, _DIM), lambda i: (i, 0)),
            pl.BlockSpec((1, _RH), lambda i: (0, 0)),
            pl.BlockSpec((256, 4096), lambda i: (0, 0)),
            pl.BlockSpec((_RH, 1024), lambda i: (0, 0)),
        ],
        out_specs=pl.BlockSpec((_EBLK, _DIM), lambda i: (i, 0)),
        out_shape=jax.ShapeDtypeStruct((edges, _DIM), jnp.float32),
    )(attr2, hsrc, rad_w1, w_big, r_rep)


def _scatter_call(fdst2, msg, zeros_hbm, chunk, nchunks):
    edges = fdst2.shape[0] * fdst2.shape[1] // nchunks
    bn = zeros_hbm.shape[0]
    per_w = edges // (_NC * _NS)
    rows_per_w = fdst2.shape[1]
    k = per_w // rows_per_w
    base_row = chunk * (edges // rows_per_w)
    rows_per_tile = bn // _NS
    mesh = plsc.VectorSubcoreMesh(core_axis_name="c", subcore_axis_name="s",
                                  num_cores=_NC, num_subcores=_NS)

    @functools.partial(
        pl.kernel,
        out_type=jax.ShapeDtypeStruct((_NC, bn, _DIM), jnp.float32),
        mesh=mesh,
        scratch_types=[
            pltpu.VMEM((k, rows_per_w), jnp.int32),
            pltpu.VMEM((per_w, _DIM), jnp.float32),
            pltpu.VMEM_SHARED((bn, _DIM), jnp.float32),
        ],
    )
    def _scatter(idx_hbm, msg_hbm, z_hbm, out_hbm, idx_v, rows_v, acc):
        cid = lax.axis_index("c")
        sid = lax.axis_index("s")
        wid = sid * _NC + cid
        r0 = sid * rows_per_tile
        pltpu.sync_copy(z_hbm.at[pl.ds(r0, rows_per_tile)],
                        acc.at[pl.ds(r0, rows_per_tile)])
        pltpu.sync_copy(idx_hbm.at[pl.ds(base_row + wid * k, k)], idx_v)
        pltpu.sync_copy(msg_hbm.at[pl.ds(wid * per_w, per_w)], rows_v)
        plsc.subcore_barrier()
        for j in range(k):
            pltpu.sync_copy(rows_v.at[pl.ds(j * rows_per_w, rows_per_w)],
                            acc.at[idx_v.at[j]], add=True)
        plsc.subcore_barrier()
        pltpu.sync_copy(acc.at[pl.ds(r0, rows_per_tile)],
                        out_hbm.at[cid, pl.ds(r0, rows_per_tile)])

    return _scatter(fdst2, msg, zeros_hbm)


def _fin_body(*refs):
    part_refs = refs[:-7]
    h_ref, p_ref, ws_ref, bs_ref, wv_ref, pout_ref, out_ref = refs[-7:]
    hp = jnp.dot(h_ref[...], p_ref[...], preferred_element_type=jnp.float32)
    res = _equiv_planar(hp, ws_ref[...], bs_ref[...], wv_ref[...])
    x = res
    for pr in part_refs:
        x = x + pr[0] + pr[1]
    s = x[:, 0:_MUL]
    v0 = x[:, _MUL:2 * _MUL]
    v1 = x[:, 2 * _MUL:3 * _MUL]
    v2 = x[:, 3 * _MUL:4 * _MUL]
    ns = jnp.abs(s)
    s2 = s * (ns * jax.nn.sigmoid(ns)) / (ns + 1e-8)
    nv = jnp.sqrt(v0 * v0 + v1 * v1 + v2 * v2)
    f = (nv * jax.nn.sigmoid(nv)) / (nv + 1e-8)
    y = jnp.concatenate([s2, v0 * f, v1 * f, v2 * f], axis=1)
    out_ref[...] = jnp.dot(y, pout_ref[...],
                           preferred_element_type=jnp.float32)


def _fin_call(parts_list, h2, p_in, res_ws, res_bs, res_wv, pout):
    bn = h2.shape[0]
    return pl.pallas_call(
        _fin_body,
        out_shape=jax.ShapeDtypeStruct((bn, _DIM), jnp.float32),
    )(*parts_list, h2, p_in, res_ws, res_bs, res_wv, pout)


def kernel(h, e_src, e_dst, e_attr3, li_ws, li_bs, li_wv,
           rad_w1, rad_w2, res_ws, res_bs, res_wv):
    b, n, c = h.shape
    e = e_attr3.shape[1]
    edges = b * e

    p_in = jnp.asarray(_P_PLANAR)
    rad4 = rad_w2.reshape(_RH, 4, _MUL, _MUL)
    def perm(p):
        return rad4[:, p].transpose(1, 0, 2).reshape(_MUL, _RH * _MUL)
    z = jnp.zeros((_MUL, _RH * _MUL), jnp.float32)
    w_mv = [perm(2), perm(1)]
    rows = []
    for blk_i, pair in enumerate([[perm(0), perm(3)], w_mv, w_mv, w_mv]):
        for half in range(2):
            rows.append(jnp.concatenate(
                [pair[half] if cb == blk_i else z for cb in range(4)], axis=1))
    w_big = jnp.concatenate(rows, axis=0).astype(jnp.bfloat16)
    r_rep = jnp.asarray(np.repeat(np.eye(_RH, dtype=np.float32), _MUL, axis=1))
    pout = jnp.asarray(_P_OUT)
    zeros = jnp.zeros((b * n, _DIM), jnp.float32)

    h2 = h.reshape(b * n, c)
    hin, fsrc, fdst = _pre_call(h2, p_in, li_ws, li_bs.reshape(1, _MUL),
                                li_wv, e_src, e_dst)
    idx_cols = 128
    fsrc2 = fsrc.reshape(edges // idx_cols, idx_cols)
    fdst2 = fdst.reshape(edges // idx_cols, idx_cols)
    attr2 = e_attr3.transpose(2, 0, 1).reshape(3, edges)
    nchunks = 1
    parts_list = []
    for ck in range(nchunks):
        hsrc = _gather_call(fsrc2, hin, ck, nchunks)
        msg = _msg_call(attr2, hsrc, rad_w1, w_big, r_rep,
                        ck, nchunks)
        parts_list.append(_scatter_call(fdst2, msg, zeros, ck, nchunks))
    out = _fin_call(parts_list, h2, p_in, res_ws, res_bs.reshape(1, _MUL),
                    res_wv, pout)
    return out.reshape(b, n, c)
